# baseline (speedup 1.0000x reference)
import functools
import math

import jax
import jax.numpy as jnp
from jax import lax
from jax.experimental import pallas as pl
from jax.experimental.pallas import tpu as pltpu
from jax.experimental.pallas import tpu_sc as plsc

N_NODES = 50000
N_PATHS = 100000
N_ENTRIES = 1600000
T = 6
F = 4

NC, NS = 2, 16
E_PAD = 1638400
CHUNK = 10240
N_TAB = 51200
P_PAD = 100352
H = P_PAD // 2
DUMMY = 1024
A_ROWS = H + DUMMY
OUT_PER_SUB = H // NS


def _table_body(x_ref, o_ref):
    inv = 1.0 / x_ref[0:1, :]
    r = jnp.exp(-inv)
    hs = []
    p = inv
    for _t in range(T):
        hs.append(p)
        p = p * r
    rows = []
    angs = []
    for f in range(F):
        re = None
        im = None
        for t in range(T):
            c = math.cos(2.0 * math.pi * f * t / T)
            s = -math.sin(2.0 * math.pi * f * t / T)
            term_re = hs[t] * c if c != 0.0 else None
            term_im = hs[t] * s if s != 0.0 else None
            if term_re is not None:
                re = term_re if re is None else re + term_re
            if term_im is not None:
                im = term_im if im is None else im + term_im
        if im is None:
            im = jnp.zeros_like(re)
        rows.append(jnp.log(jnp.sqrt(re * re + im * im) + 1e-12))
        angs.append(jnp.arctan2(im, re))
    o_ref[:, :] = jnp.concatenate(rows + angs, axis=0)


def _build_table(x):
    L = 2048
    grid = N_TAB // L
    xt = jnp.pad(x[:, 0], (0, N_TAB - N_NODES),
                 constant_values=1.0).reshape(1, N_TAB)
    tab_t = pl.pallas_call(
        _table_body,
        grid=(grid,),
        in_specs=[pl.BlockSpec((1, L), lambda i: (0, i))],
        out_specs=pl.BlockSpec((8, L), lambda i: (0, i)),
        out_shape=jax.ShapeDtypeStruct((8, N_TAB), jnp.float32),
    )(xt)
    return tab_t.T


def _seg_body(table, nodes, lidx, zeros, bounds, out, nodes_v, idxs_v,
              rows_v, bnd_v, acc, sem):
    c = lax.axis_index("c")
    s = lax.axis_index("s")
    pltpu.sync_copy(bounds.at[pl.ds(c * 32, 32)], bnd_v)
    bvec0 = bnd_v[pl.ds(0, 16)]
    bvec1 = bnd_v[pl.ds(16, 16)]
    start = bvec0[0]
    stop = bvec1[0]
    pltpu.sync_copy(zeros.at[pl.ds(s * OUT_PER_SUB, OUT_PER_SUB)],
                    acc.at[pl.ds(s * OUT_PER_SUB, OUT_PER_SUB)])
    plsc.subcore_barrier()

    @pl.loop(start + s, stop, step=NS)
    def _chunk(k):
        off = k * CHUNK
        pltpu.sync_copy(nodes.at[pl.ds(off, CHUNK)], nodes_v)
        pltpu.sync_copy(lidx.at[pl.ds(c * E_PAD + off, CHUNK)], idxs_v)
        pltpu.async_copy(table.at[nodes_v], rows_v, sem).wait()
        pltpu.sync_copy(rows_v, acc.at[idxs_v], add=True)

    plsc.subcore_barrier()
    pltpu.sync_copy(acc.at[pl.ds(s * OUT_PER_SUB, OUT_PER_SUB)],
                    out.at[pl.ds(c * H + s * OUT_PER_SUB, OUT_PER_SUB)])


@functools.cache
def _seg_call():
    return pl.kernel(
        _seg_body,
        mesh=plsc.VectorSubcoreMesh(core_axis_name="c", subcore_axis_name="s",
                                    num_cores=NC),
        compiler_params=pltpu.CompilerParams(use_tc_tiling_on_sc=False),
        out_type=jax.ShapeDtypeStruct((P_PAD, 8), jnp.float32),
        scratch_types=[
            pltpu.VMEM((CHUNK,), jnp.int32),
            pltpu.VMEM((CHUNK,), jnp.int32),
            pltpu.VMEM((CHUNK, 8), jnp.float32),
            pltpu.VMEM((32,), jnp.int32),
            pltpu.VMEM_SHARED((A_ROWS, 8), jnp.float32),
            pltpu.SemaphoreType.DMA,
        ],
    )


def _post_body(p_ref, o_ref):
    mag = [jnp.exp(p_ref[f:f + 1, :]) for f in range(F)]
    ang = [p_ref[F + f:F + f + 1, :] for f in range(F)]
    re = [mag[f] * jnp.cos(ang[f]) for f in range(F)]
    im = [mag[f] * jnp.sin(ang[f]) for f in range(F)]
    ys = []
    for t in range(T):
        acc = re[0] + ((-1.0) ** t) * re[3]
        for f in (1, 2):
            cf = math.cos(2.0 * math.pi * f * t / T)
            sf = math.sin(2.0 * math.pi * f * t / T)
            acc = acc + 2.0 * (re[f] * cf - im[f] * sf)
        ys.append(acc * (1.0 / T))
    zs = [jnp.maximum(ys[T - 1 - t], 0.0) for t in range(T)]
    tot = zs[0]
    for t in range(1, T):
        tot = tot + zs[t]
    inv_tot = 1.0 / tot
    o_ref[:, :] = jnp.concatenate([z * inv_tot for z in zs], axis=0)


def _postprocess(part):
    L = 2048
    grid = P_PAD // L
    out_t = pl.pallas_call(
        _post_body,
        grid=(grid,),
        in_specs=[pl.BlockSpec((8, L), lambda i: (0, i))],
        out_specs=pl.BlockSpec((6, L), lambda i: (0, i)),
        out_shape=jax.ShapeDtypeStruct((6, P_PAD), jnp.float32),
    )(part.T)
    return out_t.T


def kernel(x, path_idxs, path_nodes, coords):
    table = _build_table(x)
    pad = E_PAD - N_ENTRIES
    nodes_p = jnp.pad(path_nodes, (0, pad))
    pad_vals = N_PATHS + (jnp.arange(pad, dtype=jnp.int32) % 96)
    idxs_p = jnp.concatenate([path_idxs, pad_vals])
    dummy_rows = H + (jnp.arange(E_PAD, dtype=jnp.int32) % DUMMY)
    lidx = []
    for c in range(NC):
        loc = idxs_p - c * H
        lidx.append(jnp.where((loc >= 0) & (loc < H), loc, dummy_rows))
    lidx = jnp.concatenate(lidx)
    split = jnp.sum(path_idxs < H).astype(jnp.int32)
    nch = E_PAD // CHUNK
    c0 = (split + CHUNK - 1) // CHUNK
    f1 = split // CHUNK
    bounds = jnp.concatenate([
        jnp.zeros((16,), jnp.int32),
        jnp.full((16,), c0, jnp.int32),
        jnp.full((16,), f1, jnp.int32),
        jnp.full((16,), nch, jnp.int32),
    ])
    zeros = jnp.zeros((H, 8), jnp.float32)
    part = _seg_call()(table, nodes_p, lidx, zeros, bounds)
    out = _postprocess(part)
    return out[:N_PATHS]


# speedup vs baseline: 1.6221x; 1.6221x over previous
import functools
import math

import jax
import jax.numpy as jnp
from jax import lax
from jax.experimental import pallas as pl
from jax.experimental.pallas import tpu as pltpu
from jax.experimental.pallas import tpu_sc as plsc

N_NODES = 50000
N_PATHS = 100000
N_ENTRIES = 1600000
T = 6
F = 4

NC, NS = 2, 16
E_PAD = N_ENTRIES
CHUNK = 6400
N_TAB = 51200
P_PAD = 100352
H = P_PAD // 2
DUMMY = 1024
A_ROWS = H + DUMMY
OUT_PER_SUB = H // NS


def _table_body(x_ref, o_ref):
    inv = 1.0 / x_ref[0:1, :]
    r = jnp.exp(-inv)
    hs = []
    p = inv
    for _t in range(T):
        hs.append(p)
        p = p * r
    rows = []
    angs = []
    for f in range(F):
        re = None
        im = None
        for t in range(T):
            c = math.cos(2.0 * math.pi * f * t / T)
            s = -math.sin(2.0 * math.pi * f * t / T)
            term_re = hs[t] * c if c != 0.0 else None
            term_im = hs[t] * s if s != 0.0 else None
            if term_re is not None:
                re = term_re if re is None else re + term_re
            if term_im is not None:
                im = term_im if im is None else im + term_im
        if im is None:
            im = jnp.zeros_like(re)
        rows.append(jnp.log(jnp.sqrt(re * re + im * im) + 1e-12))
        angs.append(jnp.arctan2(im, re))
    o_ref[:, :] = jnp.concatenate(rows + angs, axis=0)


def _build_table(x):
    L = 2048
    grid = N_TAB // L
    xt = jnp.pad(x[:, 0], (0, N_TAB - N_NODES),
                 constant_values=1.0).reshape(1, N_TAB)
    tab_t = pl.pallas_call(
        _table_body,
        grid=(grid,),
        in_specs=[pl.BlockSpec((1, L), lambda i: (0, i))],
        out_specs=pl.BlockSpec((8, L), lambda i: (0, i)),
        out_shape=jax.ShapeDtypeStruct((8, N_TAB), jnp.float32),
    )(xt)
    return tab_t.T


def _seg_body(table, nodes, lidx, zeros, bounds, out, nodes_v, idxs_v,
              rows_v, bnd_v, acc, sem):
    c = lax.axis_index("c")
    s = lax.axis_index("s")
    pltpu.sync_copy(bounds.at[pl.ds(c * 32, 32)], bnd_v)
    bvec0 = bnd_v[pl.ds(0, 16)]
    bvec1 = bnd_v[pl.ds(16, 16)]
    start = bvec0[0]
    stop = bvec1[0]
    pltpu.sync_copy(zeros.at[pl.ds(s * OUT_PER_SUB, OUT_PER_SUB)],
                    acc.at[pl.ds(s * OUT_PER_SUB, OUT_PER_SUB)])
    plsc.subcore_barrier()

    @pl.loop(start + s, stop, step=NS)
    def _chunk(k):
        off = k * CHUNK
        pltpu.sync_copy(nodes.at[pl.ds(off, CHUNK)], nodes_v)
        pltpu.sync_copy(lidx.at[pl.ds(c * E_PAD + off, CHUNK)], idxs_v)
        pltpu.async_copy(table.at[nodes_v], rows_v, sem).wait()
        pltpu.sync_copy(rows_v, acc.at[idxs_v], add=True)

    plsc.subcore_barrier()
    pltpu.sync_copy(acc.at[pl.ds(s * OUT_PER_SUB, OUT_PER_SUB)],
                    out.at[pl.ds(c * H + s * OUT_PER_SUB, OUT_PER_SUB)])


@functools.cache
def _seg_call():
    return pl.kernel(
        _seg_body,
        mesh=plsc.VectorSubcoreMesh(core_axis_name="c", subcore_axis_name="s",
                                    num_cores=NC),
        compiler_params=pltpu.CompilerParams(use_tc_tiling_on_sc=False),
        out_type=jax.ShapeDtypeStruct((P_PAD, 8), jnp.float32),
        scratch_types=[
            pltpu.VMEM((CHUNK,), jnp.int32),
            pltpu.VMEM((CHUNK,), jnp.int32),
            pltpu.VMEM((CHUNK, 8), jnp.float32),
            pltpu.VMEM((32,), jnp.int32),
            pltpu.VMEM_SHARED((A_ROWS, 8), jnp.float32),
            pltpu.SemaphoreType.DMA,
        ],
    )


def _post_body(p_ref, o_ref):
    mag = [jnp.exp(p_ref[f:f + 1, :]) for f in range(F)]
    ang = [p_ref[F + f:F + f + 1, :] for f in range(F)]
    re = [mag[f] * jnp.cos(ang[f]) for f in range(F)]
    im = [mag[f] * jnp.sin(ang[f]) for f in range(F)]
    ys = []
    for t in range(T):
        acc = re[0] + ((-1.0) ** t) * re[3]
        for f in (1, 2):
            cf = math.cos(2.0 * math.pi * f * t / T)
            sf = math.sin(2.0 * math.pi * f * t / T)
            acc = acc + 2.0 * (re[f] * cf - im[f] * sf)
        ys.append(acc * (1.0 / T))
    zs = [jnp.maximum(ys[T - 1 - t], 0.0) for t in range(T)]
    tot = zs[0]
    for t in range(1, T):
        tot = tot + zs[t]
    inv_tot = 1.0 / tot
    o_ref[:, :] = jnp.concatenate([z * inv_tot for z in zs], axis=0)


def _postprocess(part):
    L = 2048
    grid = P_PAD // L
    out_t = pl.pallas_call(
        _post_body,
        grid=(grid,),
        in_specs=[pl.BlockSpec((8, L), lambda i: (0, i))],
        out_specs=pl.BlockSpec((6, L), lambda i: (0, i)),
        out_shape=jax.ShapeDtypeStruct((6, P_PAD), jnp.float32),
    )(part.T)
    return out_t.T


def kernel(x, path_idxs, path_nodes, coords):
    table = _build_table(x)
    dummy_rows = H + (jnp.arange(E_PAD, dtype=jnp.int32) % DUMMY)
    lidx = []
    for c in range(NC):
        loc = path_idxs - c * H
        lidx.append(jnp.where((loc >= 0) & (loc < H), loc, dummy_rows))
    lidx = jnp.concatenate(lidx)
    nodes_p = path_nodes
    split = jnp.sum(path_idxs < H).astype(jnp.int32)
    nch = E_PAD // CHUNK
    c0 = (split + CHUNK - 1) // CHUNK
    f1 = split // CHUNK
    bounds = jnp.concatenate([
        jnp.zeros((16,), jnp.int32),
        jnp.full((16,), c0, jnp.int32),
        jnp.full((16,), f1, jnp.int32),
        jnp.full((16,), nch, jnp.int32),
    ])
    zeros = jnp.zeros((H, 8), jnp.float32)
    part = _seg_call()(table, nodes_p, lidx, zeros, bounds)
    out = _postprocess(part)
    return out[:N_PATHS]


# speedup vs baseline: 1.7002x; 1.0481x over previous
import functools
import math

import jax
import jax.numpy as jnp
from jax import lax
from jax.experimental import pallas as pl
from jax.experimental.pallas import tpu as pltpu
from jax.experimental.pallas import tpu_sc as plsc

N_NODES = 50000
N_PATHS = 100000
N_ENTRIES = 1600000
T = 6
F = 4

NC, NS = 2, 16
E_PAD = N_ENTRIES
CHUNK = 6400
N_TAB = 51200
P_PAD = 100352
H = P_PAD // 2
DUMMY = 1024
A_ROWS = H + DUMMY
OUT_PER_SUB = H // NS


def _table_body(x_ref, o_ref):
    inv = 1.0 / x_ref[0:1, :]
    r = jnp.exp(-inv)
    hs = []
    p = inv
    for _t in range(T):
        hs.append(p)
        p = p * r
    rows = []
    angs = []
    for f in range(F):
        re = None
        im = None
        for t in range(T):
            c = math.cos(2.0 * math.pi * f * t / T)
            s = -math.sin(2.0 * math.pi * f * t / T)
            term_re = hs[t] * c if c != 0.0 else None
            term_im = hs[t] * s if s != 0.0 else None
            if term_re is not None:
                re = term_re if re is None else re + term_re
            if term_im is not None:
                im = term_im if im is None else im + term_im
        if im is None:
            im = jnp.zeros_like(re)
        rows.append(jnp.log(jnp.sqrt(re * re + im * im) + 1e-12))
        angs.append(jnp.arctan2(im, re))
    o_ref[:, :] = jnp.concatenate(rows + angs, axis=0).T


def _build_table(x):
    L = 2048
    grid = N_TAB // L
    xt = jnp.pad(x[:, 0], (0, N_TAB - N_NODES),
                 constant_values=1.0).reshape(1, N_TAB)
    return pl.pallas_call(
        _table_body,
        grid=(grid,),
        in_specs=[pl.BlockSpec((1, L), lambda i: (0, i))],
        out_specs=pl.BlockSpec((L, 8), lambda i: (i, 0)),
        out_shape=jax.ShapeDtypeStruct((N_TAB, 8), jnp.float32),
    )(xt)


def _seg_body(table, nodes, lidx, zeros, bounds, out, nodes_v, idxs_v,
              rows_v, bnd_v, acc, sem):
    c = lax.axis_index("c")
    s = lax.axis_index("s")
    pltpu.sync_copy(bounds.at[pl.ds(c * 32, 32)], bnd_v)
    bvec0 = bnd_v[pl.ds(0, 16)]
    bvec1 = bnd_v[pl.ds(16, 16)]
    start = bvec0[0]
    stop = bvec1[0]
    pltpu.sync_copy(zeros.at[pl.ds(s * OUT_PER_SUB, OUT_PER_SUB)],
                    acc.at[pl.ds(s * OUT_PER_SUB, OUT_PER_SUB)])
    plsc.subcore_barrier()

    @pl.loop(start + s, stop, step=NS)
    def _chunk(k):
        off = k * CHUNK
        pltpu.sync_copy(nodes.at[pl.ds(off, CHUNK)], nodes_v)
        pltpu.sync_copy(lidx.at[pl.ds(c * E_PAD + off, CHUNK)], idxs_v)
        pltpu.async_copy(table.at[nodes_v], rows_v, sem).wait()
        pltpu.sync_copy(rows_v, acc.at[idxs_v], add=True)

    plsc.subcore_barrier()
    pltpu.sync_copy(acc.at[pl.ds(s * OUT_PER_SUB, OUT_PER_SUB)],
                    out.at[pl.ds(c * H + s * OUT_PER_SUB, OUT_PER_SUB)])


@functools.cache
def _seg_call():
    return pl.kernel(
        _seg_body,
        mesh=plsc.VectorSubcoreMesh(core_axis_name="c", subcore_axis_name="s",
                                    num_cores=NC),
        compiler_params=pltpu.CompilerParams(use_tc_tiling_on_sc=False),
        out_type=jax.ShapeDtypeStruct((P_PAD, 8), jnp.float32),
        scratch_types=[
            pltpu.VMEM((CHUNK,), jnp.int32),
            pltpu.VMEM((CHUNK,), jnp.int32),
            pltpu.VMEM((CHUNK, 8), jnp.float32),
            pltpu.VMEM((32,), jnp.int32),
            pltpu.VMEM_SHARED((A_ROWS, 8), jnp.float32),
            pltpu.SemaphoreType.DMA,
        ],
    )


def _post_body(p_ref, o_ref):
    pt = p_ref[:, :].T
    mag = [jnp.exp(pt[f:f + 1, :]) for f in range(F)]
    ang = [pt[F + f:F + f + 1, :] for f in range(F)]
    re = [mag[f] * jnp.cos(ang[f]) for f in range(F)]
    im = [mag[f] * jnp.sin(ang[f]) for f in range(F)]
    ys = []
    for t in range(T):
        acc = re[0] + ((-1.0) ** t) * re[3]
        for f in (1, 2):
            cf = math.cos(2.0 * math.pi * f * t / T)
            sf = math.sin(2.0 * math.pi * f * t / T)
            acc = acc + 2.0 * (re[f] * cf - im[f] * sf)
        ys.append(acc * (1.0 / T))
    zs = [jnp.maximum(ys[T - 1 - t], 0.0) for t in range(T)]
    tot = zs[0]
    for t in range(1, T):
        tot = tot + zs[t]
    inv_tot = 1.0 / tot
    o_ref[:, :] = jnp.concatenate([z * inv_tot for z in zs], axis=0)


def _postprocess(part):
    L = 2048
    grid = P_PAD // L
    out_t = pl.pallas_call(
        _post_body,
        grid=(grid,),
        in_specs=[pl.BlockSpec((L, 8), lambda i: (i, 0))],
        out_specs=pl.BlockSpec((6, L), lambda i: (0, i)),
        out_shape=jax.ShapeDtypeStruct((6, P_PAD), jnp.float32),
    )(part)
    return out_t.T


def kernel(x, path_idxs, path_nodes, coords):
    table = _build_table(x)
    dummy_rows = H + (jnp.arange(E_PAD, dtype=jnp.int32) % DUMMY)
    lidx = []
    for c in range(NC):
        loc = path_idxs - c * H
        lidx.append(jnp.where((loc >= 0) & (loc < H), loc, dummy_rows))
    lidx = jnp.concatenate(lidx)
    nodes_p = path_nodes
    split = jnp.sum(path_idxs < H).astype(jnp.int32)
    nch = E_PAD // CHUNK
    c0 = (split + CHUNK - 1) // CHUNK
    f1 = split // CHUNK
    bounds = jnp.concatenate([
        jnp.zeros((16,), jnp.int32),
        jnp.full((16,), c0, jnp.int32),
        jnp.full((16,), f1, jnp.int32),
        jnp.full((16,), nch, jnp.int32),
    ])
    zeros = jnp.zeros((H, 8), jnp.float32)
    part = _seg_call()(table, nodes_p, lidx, zeros, bounds)
    out = _postprocess(part)
    return out[:N_PATHS]
